# Initial kernel scaffold; baseline (speedup 1.0000x reference)
import jax
import jax.numpy as jnp
from jax.experimental import pallas as pl


def kernel(hidden_states, attention_weights):
    raise NotImplementedError("write your pallas kernel here")


# speedup vs baseline: 1.1780x; 1.1780x over previous
import functools
import math

import jax
import jax.numpy as jnp
from jax.experimental import pallas as pl
from jax.experimental.pallas import tpu as pltpu

KEEP_RATIO = 0.5
MIN_TOKENS = 1


def _score_body(aw_ref, scores_ref, acc_ref):
    h = pl.program_id(1)
    acc_ref[h, :] = jnp.sum(aw_ref[0, 0], axis=0)

    @pl.when(h == pl.num_programs(1) - 1)
    def _():
        scores_ref[0, 0, :] = jnp.mean(acc_ref[...], axis=0)


def _prune_body(k, scores_ref, hs_ref, out_ref, mask_ref):
    s = scores_ref[0, 0, :]
    T = s.shape[0]
    s_i = s[:, None]
    s_j = s[None, :]
    i_idx = jax.lax.broadcasted_iota(jnp.int32, (T, T), 0)
    j_idx = jax.lax.broadcasted_iota(jnp.int32, (T, T), 1)
    beats = (s_j > s_i) | ((s_j == s_i) & (j_idx < i_idx))
    rank = jnp.sum(beats.astype(jnp.int32), axis=1)
    pos = jax.lax.broadcasted_iota(jnp.int32, (T,), 0)
    keep = (rank < k) | (pos == 0)
    mask_ref[0, 0, :] = keep.astype(jnp.int32)
    out_ref[0] = hs_ref[0] * keep.astype(hs_ref.dtype)[:, None]


@jax.jit
def kernel(hidden_states, attention_weights):
    B, T, D = hidden_states.shape
    _, H, _, _ = attention_weights.shape
    k = min(max(MIN_TOKENS, math.ceil(KEEP_RATIO * T)), T)

    scores = pl.pallas_call(
        _score_body,
        grid=(B, H),
        in_specs=[pl.BlockSpec((1, 1, T, T), lambda b, h: (b, h, 0, 0))],
        out_specs=pl.BlockSpec((1, 1, T), lambda b, h: (b, 0, 0)),
        out_shape=jax.ShapeDtypeStruct((B, 1, T), jnp.float32),
        scratch_shapes=[pltpu.VMEM((H, T), jnp.float32)],
        compiler_params=pltpu.CompilerParams(
            dimension_semantics=("arbitrary", "arbitrary"),
        ),
    )(attention_weights)

    pruned, mask_i32 = pl.pallas_call(
        functools.partial(_prune_body, k),
        grid=(B,),
        in_specs=[
            pl.BlockSpec((1, 1, T), lambda b: (b, 0, 0)),
            pl.BlockSpec((1, T, D), lambda b: (b, 0, 0)),
        ],
        out_specs=[
            pl.BlockSpec((1, T, D), lambda b: (b, 0, 0)),
            pl.BlockSpec((1, 1, T), lambda b: (b, 0, 0)),
        ],
        out_shape=[
            jax.ShapeDtypeStruct((B, T, D), hidden_states.dtype),
            jax.ShapeDtypeStruct((B, 1, T), jnp.int32),
        ],
    )(scores, hidden_states)

    return (pruned, mask_i32.reshape(B, T).astype(bool))
